# Initial kernel scaffold; baseline (speedup 1.0000x reference)
import jax
import jax.numpy as jnp
from jax.experimental import pallas as pl


def kernel(input_ids, attention_mask, img_embeddings, pos, emb_table, embed_scale):
    raise NotImplementedError("write your pallas kernel here")


# speedup vs baseline: 1.4658x; 1.4658x over previous
import functools

import jax
import jax.numpy as jnp
from jax import lax
from jax.experimental import pallas as pl
from jax.experimental.pallas import tpu as pltpu
from jax.experimental.pallas import tpu_sc as plsc

HID = 768
GRP = HID // 16
NC, NS = 2, 16
NW = NC * NS
RPW = 1028
CH = 64
NCH = 16
WSTR = 1032


def _build_call():
    mesh = plsc.VectorSubcoreMesh(core_axis_name="c", subcore_axis_name="s")

    @functools.partial(
        pl.kernel,
        out_type=jax.ShapeDtypeStruct((NW * RPW, HID), jnp.float32),
        mesh=mesh,
        scratch_types=[
            pltpu.VMEM((WSTR,), jnp.int32),
            pltpu.VMEM((CH, HID), jnp.float32),
            pltpu.VMEM((CH, HID), jnp.float32),
            pltpu.VMEM((NCH, CH), jnp.int32),
            pltpu.VMEM((8,), jnp.int32),
            pltpu.VMEM((8, HID), jnp.float32),
            pltpu.VMEM((8,), jnp.int32),
            pltpu.VMEM((16,), jnp.float32),
            pltpu.SemaphoreType.DMA,
            pltpu.SemaphoreType.DMA,
            pltpu.SemaphoreType.DMA,
            pltpu.SemaphoreType.DMA,
        ],
    )
    def emb_kernel(gidx_hbm, didx_hbm, dtail_hbm, table_hbm, img_hbm,
                   imgrows_hbm, scale_hbm, out_hbm, idx_v, buf0, buf1,
                   didx_v, dtail_v, simg, iidx, sv_ref,
                   sem0, sem1, wsem0, wsem1):
        cid = lax.axis_index("c")
        sid = lax.axis_index("s")
        wid = cid * NS + sid
        base = pl.multiple_of(wid * WSTR, 8)

        pltpu.sync_copy(gidx_hbm.at[pl.ds(base, WSTR)], idx_v)
        pltpu.sync_copy(scale_hbm, sv_ref)
        pltpu.sync_copy(didx_hbm.at[wid], didx_v)
        pltpu.sync_copy(dtail_hbm.at[pl.ds(wid * 8, 8)], dtail_v)
        pltpu.sync_copy(imgrows_hbm.at[pl.ds(wid * 8, 8)], iidx)
        pltpu.sync_copy(img_hbm.at[wid], simg)
        sv = sv_ref[...]

        bufs = (buf0, buf1)
        sems = (sem0, sem1)
        wsems = (wsem0, wsem1)
        gh = [None, None]
        wh = [None, None]

        def start_gather(c, b):
            n = CH if c < NCH else 8
            gh[b] = pltpu.async_copy(
                table_hbm.at[idx_v.at[pl.ds(c * CH, n)]],
                bufs[b].at[pl.ds(0, n)],
                sems[b],
            )

        def start_scatter(c, b):
            if c < NCH:
                wh[b] = pltpu.async_copy(
                    bufs[b], out_hbm.at[didx_v.at[c]], wsems[b])
            else:
                wh[b] = pltpu.async_copy(
                    bufs[b].at[pl.ds(0, 8)], out_hbm.at[dtail_v], wsems[b])

        def scale_rows(b, n):
            def body(r, carry):
                for k in range(GRP):
                    sl = pl.ds(k * 16, 16)
                    bufs[b][r, sl] = bufs[b][r, sl] * sv
                return carry
            lax.fori_loop(0, n, body, 0)

        start_gather(0, 0)
        for c in range(NCH + 1):
            b = c % 2
            if wh[1 - b] is not None:
                wh[1 - b].wait()
            if c + 1 <= NCH:
                start_gather(c + 1, 1 - b)
            gh[b].wait()
            scale_rows(b, CH if c < NCH else 8)
            start_scatter(c, b)
        wh[0].wait()

        pltpu.async_copy(simg, out_hbm.at[iidx], wsem1).wait()

    return emb_kernel


@functools.cache
def _get_call():
    return _build_call()


def kernel(input_ids, attention_mask, img_embeddings, pos, emb_table,
           embed_scale):
    ids = input_ids.astype(jnp.int32)
    bs, nshot, sl = ids.shape
    npair = bs * nshot
    posi = jnp.asarray(pos, jnp.int32)

    t = jnp.arange(sl + 1, dtype=jnp.int32)
    src = jnp.clip(jnp.where(t < posi, t, t - 1), 0, sl - 1)
    g = jnp.take(ids, src, axis=2).reshape(NW, RPW)
    gidx = jnp.concatenate([g, g[:, NCH * CH:RPW]], axis=1).reshape(-1)

    w = jnp.arange(NW, dtype=jnp.int32)[:, None]
    d_main = (w * RPW + jnp.arange(NCH * CH, dtype=jnp.int32)[None, :]
              ).reshape(NW, NCH, CH)
    tail4 = jnp.arange(4, dtype=jnp.int32)
    d_tail = (w * RPW + NCH * CH
              + jnp.concatenate([tail4, tail4])[None, :]).reshape(-1)

    ir = (jnp.arange(npair, dtype=jnp.int32) * (sl + 1) + posi).reshape(NW, 4)
    imgrows = jnp.concatenate([ir, ir], axis=1).reshape(-1)

    img4 = img_embeddings.reshape(NW, 4, HID).astype(jnp.float32)
    img_dup = jnp.concatenate([img4, img4], axis=1)
    scale_arr = jnp.full((16,), embed_scale, jnp.float32)

    out = _get_call()(
        gidx, d_main, d_tail, emb_table, img_dup, imgrows, scale_arr)
    out = out.reshape(bs, nshot, sl + 1, HID)
    mask = jnp.pad(attention_mask, ((0, 0), (0, 0), (1, 0)),
                   constant_values=1)
    return out, mask
